# baseline (speedup 1.0000x reference)
import functools

import jax
import jax.numpy as jnp
from jax import lax
from jax.experimental import pallas as pl
from jax.experimental.pallas import tpu as pltpu
from jax.experimental.pallas import tpu_sc as plsc

SC_CORES = 2
SC_SUBCORES = 16
SC_WORKERS = SC_CORES * SC_SUBCORES
SC_BLK = 768
SC_STEPS_PER_WORKER = 28
SC_N = SC_WORKERS * SC_BLK * SC_STEPS_PER_WORKER
ROW_GROUP = 8

TC_BLK = 32768


def _sc_partials(x, w_row):
    rows = x.shape[0]
    mesh = plsc.VectorSubcoreMesh(
        core_axis_name="core", subcore_axis_name="subcore")

    @functools.partial(
        pl.kernel,
        out_type=jax.ShapeDtypeStruct((rows, 128 * SC_WORKERS), jnp.float32),
        mesh=mesh,
        scratch_types=[pltpu.VMEM((rows, 128), jnp.float32)],
    )
    def sc_kernel(x_hbm, w_hbm, o_hbm, acc_vmem):
        wid = (lax.axis_index("subcore") * SC_CORES
               + lax.axis_index("core"))

        @pl.loop(0, rows)
        def _zero(r):
            for c in range(8):
                acc_vmem[pl.ds(r, 1), pl.ds(c * 16, 16)] = jnp.zeros(
                    (1, 16), jnp.float32)

        def body(x_vmem, w_vmem):
            for rg in range(0, rows, ROW_GROUP):
                def chunk(i, accs):
                    w16 = w_vmem[pl.ds(0, 1), pl.ds(i * 16, 16)]
                    return tuple(
                        accs[j]
                        + x_vmem[pl.ds(rg + j, 1), pl.ds(i * 16, 16)] * w16
                        for j in range(ROW_GROUP))

                init = tuple(jnp.zeros((1, 16), jnp.float32)
                             for _ in range(ROW_GROUP))
                accs = lax.fori_loop(0, SC_BLK // 16, chunk, init,
                                     unroll=2)
                for j in range(ROW_GROUP):
                    acc_vmem[pl.ds(rg + j, 1), pl.ds(0, 16)] += accs[j]

        pltpu.emit_pipeline(
            body,
            grid=(SC_WORKERS * SC_STEPS_PER_WORKER,),
            in_specs=[
                pl.BlockSpec((rows, SC_BLK), lambda i: (0, i)),
                pl.BlockSpec((1, SC_BLK), lambda i: (0, i)),
            ],
            out_specs=[],
            core_axis_name=("core", "subcore"),
            dimension_semantics=(pltpu.PARALLEL,),
        )(x_hbm, w_hbm)

        pltpu.sync_copy(acc_vmem, o_hbm.at[:, pl.ds(wid * 128, 128)])

    return sc_kernel(x, w_row)


def _tc_body(n_cols, col0_blocks, x_ref, w_ref, o_ref, acc_ref):
    pid = pl.program_id(0)
    last = pl.num_programs(0) - 1

    @pl.when(pid == 0)
    def _init():
        acc_ref[...] = jnp.zeros_like(acc_ref)

    def _lane_group_sum(prod):
        parts = [prod[:, g * 128:(g + 1) * 128] for g in range(TC_BLK // 128)]
        while len(parts) > 1:
            parts = [a + b for a, b in zip(parts[0::2], parts[1::2])] + (
                [parts[-1]] if len(parts) % 2 else [])
        return parts[0]

    @pl.when(pid != last)
    def _main():
        acc_ref[...] += _lane_group_sum(x_ref[...] * w_ref[...])

    @pl.when(pid == last)
    def _fini():
        x = x_ref[...]
        w = w_ref[...]
        col = (pid + col0_blocks) * TC_BLK + jax.lax.broadcasted_iota(
            jnp.int32, x.shape, 1)
        acc_ref[...] += _lane_group_sum(jnp.where(col < n_cols, x * w, 0.0))
        o_ref[...] = acc_ref[...]


def _tc_partials(x, w_row):
    rows, n = x.shape
    col0_blocks = SC_N // TC_BLK
    grid = pl.cdiv(n - SC_N, TC_BLK)
    return pl.pallas_call(
        functools.partial(_tc_body, n, col0_blocks),
        grid=(grid,),
        in_specs=[
            pl.BlockSpec((rows, TC_BLK), lambda i: (0, i + col0_blocks)),
            pl.BlockSpec((1, TC_BLK), lambda i: (0, i + col0_blocks)),
        ],
        out_specs=pl.BlockSpec((rows, 128), lambda i: (0, 0)),
        out_shape=jax.ShapeDtypeStruct((rows, 128), jnp.float32),
        scratch_shapes=[pltpu.VMEM((rows, 128), jnp.float32)],
        compiler_params=pltpu.CompilerParams(
            dimension_semantics=("arbitrary",)),
    )(x, w_row)


def _fin_body(sc_ref, tc_ref, b_ref, o_ref):
    s = sc_ref[...].sum(axis=1, keepdims=True)
    t = tc_ref[...].sum(axis=1, keepdims=True)
    o_ref[...] = s + t + b_ref[0]


def _finalize(sc_parts, tc_parts, b_arr):
    rows = sc_parts.shape[0]
    return pl.pallas_call(
        _fin_body,
        in_specs=[
            pl.BlockSpec(sc_parts.shape, lambda: (0, 0)),
            pl.BlockSpec(tc_parts.shape, lambda: (0, 0)),
            pl.BlockSpec(memory_space=pltpu.SMEM),
        ],
        out_specs=pl.BlockSpec((rows, 1), lambda: (0, 0)),
        out_shape=jax.ShapeDtypeStruct((rows, 1), jnp.float32),
    )(sc_parts, tc_parts, b_arr)


def kernel(inputs, W, b):
    rows, n = inputs.shape
    w_row = W.reshape(1, n)
    b_arr = jnp.asarray(b, jnp.float32).reshape(1)
    sc_parts = _sc_partials(inputs, w_row)
    tc_parts = _tc_partials(inputs, w_row)
    return _finalize(sc_parts, tc_parts, b_arr)


# speedup vs baseline: 1.0189x; 1.0189x over previous
import functools

import jax
import jax.numpy as jnp
from jax import lax
from jax.experimental import pallas as pl
from jax.experimental.pallas import tpu as pltpu
from jax.experimental.pallas import tpu_sc as plsc

SC_CORES = 2
SC_SUBCORES = 16
SC_WORKERS = SC_CORES * SC_SUBCORES
SC_BLK = 768
SC_STEPS_PER_WORKER = 20
SC_N = SC_WORKERS * SC_BLK * SC_STEPS_PER_WORKER
ROW_GROUP = 16

TC_BLK = 32768


def _sc_partials(x, w_row):
    rows = x.shape[0]
    mesh = plsc.VectorSubcoreMesh(
        core_axis_name="core", subcore_axis_name="subcore")

    @functools.partial(
        pl.kernel,
        out_type=jax.ShapeDtypeStruct((rows, 128 * SC_WORKERS), jnp.float32),
        mesh=mesh,
        scratch_types=[pltpu.VMEM((rows, 128), jnp.float32)],
    )
    def sc_kernel(x_hbm, w_hbm, o_hbm, acc_vmem):
        wid = (lax.axis_index("subcore") * SC_CORES
               + lax.axis_index("core"))

        @pl.loop(0, rows)
        def _zero(r):
            for c in range(8):
                acc_vmem[pl.ds(r, 1), pl.ds(c * 16, 16)] = jnp.zeros(
                    (1, 16), jnp.float32)

        def body(x_vmem, w_vmem):
            for rg in range(0, rows, ROW_GROUP):
                def chunk(i, accs):
                    w16 = w_vmem[pl.ds(0, 1), pl.ds(i * 16, 16)]
                    return tuple(
                        accs[j]
                        + x_vmem[pl.ds(rg + j, 1), pl.ds(i * 16, 16)] * w16
                        for j in range(ROW_GROUP))

                init = tuple(jnp.zeros((1, 16), jnp.float32)
                             for _ in range(ROW_GROUP))
                accs = lax.fori_loop(0, SC_BLK // 16, chunk, init,
                                     unroll=4)
                for j in range(ROW_GROUP):
                    acc_vmem[pl.ds(rg + j, 1), pl.ds(0, 16)] += accs[j]

        pltpu.emit_pipeline(
            body,
            grid=(SC_WORKERS * SC_STEPS_PER_WORKER,),
            in_specs=[
                pl.BlockSpec((rows, SC_BLK), lambda i: (0, i)),
                pl.BlockSpec((1, SC_BLK), lambda i: (0, i)),
            ],
            out_specs=[],
            core_axis_name=("core", "subcore"),
            dimension_semantics=(pltpu.PARALLEL,),
        )(x_hbm, w_hbm)

        pltpu.sync_copy(acc_vmem, o_hbm.at[:, pl.ds(wid * 128, 128)])

    return sc_kernel(x, w_row)


def _tc_body(n_cols, col0_blocks, x_ref, w_ref, o_ref, acc_ref):
    pid = pl.program_id(0)
    last = pl.num_programs(0) - 1

    @pl.when(pid == 0)
    def _init():
        acc_ref[...] = jnp.zeros_like(acc_ref)

    def _lane_group_sum(prod):
        parts = [prod[:, g * 128:(g + 1) * 128] for g in range(TC_BLK // 128)]
        while len(parts) > 1:
            parts = [a + b for a, b in zip(parts[0::2], parts[1::2])] + (
                [parts[-1]] if len(parts) % 2 else [])
        return parts[0]

    @pl.when(pid != last)
    def _main():
        acc_ref[...] += _lane_group_sum(x_ref[...] * w_ref[...])

    @pl.when(pid == last)
    def _fini():
        x = x_ref[...]
        w = w_ref[...]
        col = (pid + col0_blocks) * TC_BLK + jax.lax.broadcasted_iota(
            jnp.int32, x.shape, 1)
        acc_ref[...] += _lane_group_sum(jnp.where(col < n_cols, x * w, 0.0))
        o_ref[...] = acc_ref[...]


def _tc_partials(x, w_row):
    rows, n = x.shape
    col0_blocks = SC_N // TC_BLK
    grid = pl.cdiv(n - SC_N, TC_BLK)
    return pl.pallas_call(
        functools.partial(_tc_body, n, col0_blocks),
        grid=(grid,),
        in_specs=[
            pl.BlockSpec((rows, TC_BLK), lambda i: (0, i + col0_blocks)),
            pl.BlockSpec((1, TC_BLK), lambda i: (0, i + col0_blocks)),
        ],
        out_specs=pl.BlockSpec((rows, 128), lambda i: (0, 0)),
        out_shape=jax.ShapeDtypeStruct((rows, 128), jnp.float32),
        scratch_shapes=[pltpu.VMEM((rows, 128), jnp.float32)],
        compiler_params=pltpu.CompilerParams(
            dimension_semantics=("arbitrary",)),
    )(x, w_row)


def _fin_body(sc_ref, tc_ref, b_ref, o_ref):
    s = sc_ref[...].sum(axis=1, keepdims=True)
    t = tc_ref[...].sum(axis=1, keepdims=True)
    o_ref[...] = s + t + b_ref[0]


def _finalize(sc_parts, tc_parts, b_arr):
    rows = sc_parts.shape[0]
    return pl.pallas_call(
        _fin_body,
        in_specs=[
            pl.BlockSpec(sc_parts.shape, lambda: (0, 0)),
            pl.BlockSpec(tc_parts.shape, lambda: (0, 0)),
            pl.BlockSpec(memory_space=pltpu.SMEM),
        ],
        out_specs=pl.BlockSpec((rows, 1), lambda: (0, 0)),
        out_shape=jax.ShapeDtypeStruct((rows, 1), jnp.float32),
    )(sc_parts, tc_parts, b_arr)


def kernel(inputs, W, b):
    rows, n = inputs.shape
    w_row = W.reshape(1, n)
    b_arr = jnp.asarray(b, jnp.float32).reshape(1)
    sc_parts = _sc_partials(inputs, w_row)
    tc_parts = _tc_partials(inputs, w_row)
    return _finalize(sc_parts, tc_parts, b_arr)


# speedup vs baseline: 1.0374x; 1.0182x over previous
import functools

import jax
import jax.numpy as jnp
from jax import lax
from jax.experimental import pallas as pl
from jax.experimental.pallas import tpu as pltpu
from jax.experimental.pallas import tpu_sc as plsc

SC_CORES = 2
SC_SUBCORES = 16
SC_WORKERS = SC_CORES * SC_SUBCORES
SC_BLK = 768
SC_STEPS_PER_WORKER = 4
SC_N = SC_WORKERS * SC_BLK * SC_STEPS_PER_WORKER
ROW_GROUP = 16

TC_BLK = 32768


def _sc_partials(x, w_row):
    rows = x.shape[0]
    mesh = plsc.VectorSubcoreMesh(
        core_axis_name="core", subcore_axis_name="subcore")

    @functools.partial(
        pl.kernel,
        out_type=jax.ShapeDtypeStruct((rows, 128 * SC_WORKERS), jnp.float32),
        mesh=mesh,
        scratch_types=[pltpu.VMEM((rows, 128), jnp.float32)],
    )
    def sc_kernel(x_hbm, w_hbm, o_hbm, acc_vmem):
        wid = (lax.axis_index("subcore") * SC_CORES
               + lax.axis_index("core"))

        @pl.loop(0, rows)
        def _zero(r):
            for c in range(8):
                acc_vmem[pl.ds(r, 1), pl.ds(c * 16, 16)] = jnp.zeros(
                    (1, 16), jnp.float32)

        def body(x_vmem, w_vmem):
            for rg in range(0, rows, ROW_GROUP):
                def chunk(i, accs):
                    w16 = w_vmem[pl.ds(0, 1), pl.ds(i * 16, 16)]
                    return tuple(
                        accs[j]
                        + x_vmem[pl.ds(rg + j, 1), pl.ds(i * 16, 16)] * w16
                        for j in range(ROW_GROUP))

                init = tuple(jnp.zeros((1, 16), jnp.float32)
                             for _ in range(ROW_GROUP))
                accs = lax.fori_loop(0, SC_BLK // 16, chunk, init,
                                     unroll=4)
                for j in range(ROW_GROUP):
                    acc_vmem[pl.ds(rg + j, 1), pl.ds(0, 16)] += accs[j]

        pltpu.emit_pipeline(
            body,
            grid=(SC_WORKERS * SC_STEPS_PER_WORKER,),
            in_specs=[
                pl.BlockSpec((rows, SC_BLK), lambda i: (0, i)),
                pl.BlockSpec((1, SC_BLK), lambda i: (0, i)),
            ],
            out_specs=[],
            core_axis_name=("core", "subcore"),
            dimension_semantics=(pltpu.PARALLEL,),
        )(x_hbm, w_hbm)

        pltpu.sync_copy(acc_vmem, o_hbm.at[:, pl.ds(wid * 128, 128)])

    return sc_kernel(x, w_row)


def _tc_body(n_cols, col0_blocks, x_ref, w_ref, o_ref, acc_ref):
    pid = pl.program_id(0)
    last = pl.num_programs(0) - 1

    @pl.when(pid == 0)
    def _init():
        acc_ref[...] = jnp.zeros_like(acc_ref)

    def _lane_group_sum(prod):
        parts = [prod[:, g * 128:(g + 1) * 128] for g in range(TC_BLK // 128)]
        while len(parts) > 1:
            parts = [a + b for a, b in zip(parts[0::2], parts[1::2])] + (
                [parts[-1]] if len(parts) % 2 else [])
        return parts[0]

    @pl.when(pid != last)
    def _main():
        acc_ref[...] += _lane_group_sum(x_ref[...] * w_ref[...])

    @pl.when(pid == last)
    def _fini():
        x = x_ref[...]
        w = w_ref[...]
        col = (pid + col0_blocks) * TC_BLK + jax.lax.broadcasted_iota(
            jnp.int32, x.shape, 1)
        acc_ref[...] += _lane_group_sum(jnp.where(col < n_cols, x * w, 0.0))
        o_ref[...] = acc_ref[...]


def _tc_partials(x, w_row):
    rows, n = x.shape
    col0_blocks = SC_N // TC_BLK
    grid = pl.cdiv(n - SC_N, TC_BLK)
    return pl.pallas_call(
        functools.partial(_tc_body, n, col0_blocks),
        grid=(grid,),
        in_specs=[
            pl.BlockSpec((rows, TC_BLK), lambda i: (0, i + col0_blocks)),
            pl.BlockSpec((1, TC_BLK), lambda i: (0, i + col0_blocks)),
        ],
        out_specs=pl.BlockSpec((rows, 128), lambda i: (0, 0)),
        out_shape=jax.ShapeDtypeStruct((rows, 128), jnp.float32),
        scratch_shapes=[pltpu.VMEM((rows, 128), jnp.float32)],
        compiler_params=pltpu.CompilerParams(
            dimension_semantics=("arbitrary",)),
    )(x, w_row)


def _fin_body(sc_ref, tc_ref, b_ref, o_ref):
    s = sc_ref[...].sum(axis=1, keepdims=True)
    t = tc_ref[...].sum(axis=1, keepdims=True)
    o_ref[...] = s + t + b_ref[0]


def _finalize(sc_parts, tc_parts, b_arr):
    rows = sc_parts.shape[0]
    return pl.pallas_call(
        _fin_body,
        in_specs=[
            pl.BlockSpec(sc_parts.shape, lambda: (0, 0)),
            pl.BlockSpec(tc_parts.shape, lambda: (0, 0)),
            pl.BlockSpec(memory_space=pltpu.SMEM),
        ],
        out_specs=pl.BlockSpec((rows, 1), lambda: (0, 0)),
        out_shape=jax.ShapeDtypeStruct((rows, 1), jnp.float32),
    )(sc_parts, tc_parts, b_arr)


def kernel(inputs, W, b):
    rows, n = inputs.shape
    w_row = W.reshape(1, n)
    b_arr = jnp.asarray(b, jnp.float32).reshape(1)
    sc_parts = _sc_partials(inputs, w_row)
    tc_parts = _tc_partials(inputs, w_row)
    return _finalize(sc_parts, tc_parts, b_arr)


# speedup vs baseline: 1.2010x; 1.1577x over previous
import functools

import jax
import jax.numpy as jnp
from jax.experimental import pallas as pl
from jax.experimental.pallas import tpu as pltpu

BLK = 32768
SLABS = 4


def _mk_index_map(s):
    return lambda i: (s, i)


def _body(n_cols, *refs):
    x_refs = refs[:SLABS]
    w_ref = refs[SLABS]
    b_ref = refs[SLABS + 1]
    o_ref = refs[SLABS + 2]
    acc_ref = refs[SLABS + 3]
    pid = pl.program_id(0)
    last = pl.num_programs(0) - 1

    @pl.when(pid == 0)
    def _init():
        acc_ref[...] = jnp.zeros_like(acc_ref)

    def _lane_group_sum(prod):
        parts = [prod[:, g * 128:(g + 1) * 128] for g in range(BLK // 128)]
        while len(parts) > 1:
            parts = [a + b for a, b in zip(parts[0::2], parts[1::2])] + (
                [parts[-1]] if len(parts) % 2 else [])
        return parts[0]

    rows_per = x_refs[0].shape[0]

    @pl.when(pid != last)
    def _main():
        w = w_ref[...]
        for s in range(SLABS):
            acc_ref[s * rows_per:(s + 1) * rows_per, :] += _lane_group_sum(
                x_refs[s][...] * w)

    @pl.when(pid == last)
    def _fini():
        w = w_ref[...]
        col = pid * BLK + jax.lax.broadcasted_iota(
            jnp.int32, (rows_per, BLK), 1)
        keep = col < n_cols
        for s in range(SLABS):
            acc_ref[s * rows_per:(s + 1) * rows_per, :] += _lane_group_sum(
                jnp.where(keep, x_refs[s][...] * w, 0.0))
        o_ref[...] = acc_ref[...].sum(axis=1, keepdims=True) + b_ref[0]


def kernel(inputs, W, b):
    rows, n = inputs.shape
    grid = pl.cdiv(n, BLK)
    rows_per = rows // SLABS
    w_row = W.reshape(1, n)
    b_arr = jnp.asarray(b, jnp.float32).reshape(1)
    out = pl.pallas_call(
        functools.partial(_body, n),
        grid=(grid,),
        in_specs=(
            [pl.BlockSpec((rows_per, BLK), _mk_index_map(s))
             for s in range(SLABS)]
            + [pl.BlockSpec((1, BLK), lambda i: (0, i)),
               pl.BlockSpec(memory_space=pltpu.SMEM)]),
        out_specs=pl.BlockSpec((rows, 1), lambda i: (0, 0)),
        out_shape=jax.ShapeDtypeStruct((rows, 1), jnp.float32),
        scratch_shapes=[pltpu.VMEM((rows, 128), jnp.float32)],
        compiler_params=pltpu.CompilerParams(
            dimension_semantics=("arbitrary",)),
    )(*([inputs] * SLABS), w_row, b_arr)
    return out
